# baseline (speedup 1.0000x reference)
import functools

import jax
import jax.numpy as jnp
from jax import lax
from jax.experimental import pallas as pl
from jax.experimental.pallas import tpu as pltpu
from jax.experimental.pallas import tpu_sc as plsc

_CW = 128
_DUMP = 16


def _sc_info():
    info = plsc.get_sparse_core_info()
    return info.num_cores, info.num_subcores


def _pad_edges(senders, receivers, n):
    e = senders.shape[0]
    nc, ns = _sc_info()
    nw = nc * ns
    quantum = nw * _CW * 4
    epad = ((e + quantum - 1) // quantum) * quantum
    pad = epad - e
    if pad:
        senders = jnp.concatenate(
            [senders, jnp.zeros((pad,), jnp.int32)])
        receivers = jnp.concatenate(
            [receivers, n + (jnp.arange(pad, dtype=jnp.int32) % _DUMP)])
    return senders.reshape(-1, _CW), receivers.reshape(-1, _CW)


def _sc_segment_sum_count(nodes, send2d, recv2d):
    n, d = nodes.shape
    nchunks = send2d.shape[0]
    nc, ns = _sc_info()
    nw = nc * ns
    cpw = nchunks // nw
    hcpw = cpw // 2
    hpairs = hcpw // 2
    k = 4
    assert hcpw % 2 == 0 and hcpw % k == 0
    stripe = (n // (ns * 8)) * 8
    tail = n - ns * stripe
    nfull, rem = divmod(stripe, _CW)
    assert tail % 8 == 0 and tail + _DUMP <= _CW and rem % 8 == 0

    mesh = plsc.VectorSubcoreMesh(core_axis_name="c", subcore_axis_name="s")

    @functools.partial(
        pl.kernel,
        mesh=mesh,
        out_type=[
            jax.ShapeDtypeStruct((nc, n, d), jnp.float32),
            jax.ShapeDtypeStruct((nc, n, d), jnp.float32),
        ],
        scratch_types=[
            pltpu.VMEM((hcpw, _CW), jnp.int32),
            pltpu.VMEM((hcpw, _CW), jnp.int32),
            pltpu.VMEM((_CW, d), jnp.float32),
            pltpu.VMEM((_CW, d), jnp.float32),
            pltpu.VMEM_SHARED((n + _DUMP, d), jnp.float32),
            pltpu.SemaphoreType.DMA,
            pltpu.SemaphoreType.DMA,
        ],
    )
    def sc_kernel(nodes_hbm, send_hbm, recv_hbm, sums_out, cnts_out,
                  idx_s, idx_r, rows0, rows1, acc, sem0, sem1):
        c = lax.axis_index("c")
        s = lax.axis_index("s")
        wid = s * nc + c
        base_row = s * stripe

        def fill(buf, val):
            def fill_row(i, _):
                for j in range(d // 16):
                    buf[i, pl.ds(j * 16, 16)] = jnp.full((16,), val,
                                                         jnp.float32)
                return 0
            lax.fori_loop(0, _CW, fill_row, 0)

        def zero_acc():
            for r in range(nfull):
                pltpu.sync_copy(rows0,
                                acc.at[pl.ds(base_row + r * _CW, _CW)])
            if rem:
                pltpu.sync_copy(rows0.at[pl.ds(0, rem)],
                                acc.at[pl.ds(base_row + nfull * _CW, rem)])

            @pl.when(s == 0)
            def _zero_tail():
                pltpu.sync_copy(rows0.at[pl.ds(0, tail + _DUMP)],
                                acc.at[pl.ds(ns * stripe, tail + _DUMP)])

        def copy_out(dst):
            pltpu.sync_copy(acc.at[pl.ds(base_row, stripe)],
                            dst.at[c, pl.ds(base_row, stripe)])

            @pl.when(s == 0)
            def _copy_tail():
                pltpu.sync_copy(acc.at[pl.ds(ns * stripe, tail)],
                                dst.at[c, pl.ds(ns * stripe, tail)])

        fill(rows0, 0.0)
        zero_acc()
        plsc.subcore_barrier()

        for h in range(2):
            cbase = wid * cpw + h * hcpw
            pltpu.sync_copy(send_hbm.at[pl.ds(cbase, hcpw)], idx_s)
            pltpu.sync_copy(recv_hbm.at[pl.ds(cbase, hcpw)], idx_r)
            pltpu.async_copy(nodes_hbm.at[idx_s.at[0]], rows0, sem0)

            def pair_body(j, _):
                pltpu.make_async_copy(nodes_hbm.at[idx_s.at[0]], rows0,
                                      sem0).wait()
                pltpu.async_copy(nodes_hbm.at[idx_s.at[2 * j + 1]], rows1,
                                 sem1)
                pltpu.sync_copy(rows0, acc.at[idx_r.at[2 * j]], add=True)
                pltpu.make_async_copy(nodes_hbm.at[idx_s.at[0]], rows1,
                                      sem1).wait()

                @pl.when(j < hpairs - 1)
                def _prefetch():
                    pltpu.async_copy(nodes_hbm.at[idx_s.at[2 * j + 2]],
                                     rows0, sem0)

                pltpu.sync_copy(rows1, acc.at[idx_r.at[2 * j + 1]], add=True)
                return 0

            lax.fori_loop(0, hpairs, pair_body, 0)

        plsc.subcore_barrier()
        copy_out(sums_out)
        plsc.subcore_barrier()

        fill(rows0, 0.0)
        fill(rows1, 1.0)
        zero_acc()
        plsc.subcore_barrier()

        for h in range(2):
            cbase = wid * cpw + h * hcpw
            pltpu.sync_copy(recv_hbm.at[pl.ds(cbase, hcpw)], idx_r)

            def group_body(g, _):
                for j in range(k):
                    pltpu.async_copy(rows1, acc.at[idx_r.at[k * g + j]],
                                     sem0, add=True)
                for j in range(k):
                    pltpu.make_async_copy(rows1, acc.at[idx_r.at[0]],
                                          sem0).wait()
                return 0

            lax.fori_loop(0, hcpw // k, group_body, 0)

        plsc.subcore_barrier()
        copy_out(cnts_out)

    return sc_kernel(nodes, send2d, recv2d)


def _tc_finish(sums, cnts, nodes, w1, w2, b2):
    n, d = nodes.shape
    h = w1.shape[1]
    nc = sums.shape[0]
    rows = 2000
    grid = (n // rows,)

    def tc_kernel(sums_ref, cnts_ref, nodes_ref, w1_ref, w2_ref, b_ref, out_ref):
        ssum = sums_ref[0]
        csum = cnts_ref[0, :, 0:1]
        for i in range(1, nc):
            ssum = ssum + sums_ref[i]
            csum = csum + cnts_ref[i, :, 0:1]
        he = ssum / jnp.maximum(csum, 1.0)
        acc = jnp.dot(he, w1_ref[...], preferred_element_type=jnp.float32)
        acc = acc + jnp.dot(nodes_ref[...], w2_ref[...],
                            preferred_element_type=jnp.float32)
        out_ref[...] = jnp.maximum(acc + b_ref[...], 0.0)

    return pl.pallas_call(
        tc_kernel,
        grid=grid,
        in_specs=[
            pl.BlockSpec((nc, rows, d), lambda i: (0, i, 0)),
            pl.BlockSpec((nc, rows, d), lambda i: (0, i, 0)),
            pl.BlockSpec((rows, d), lambda i: (i, 0)),
            pl.BlockSpec((d, h), lambda i: (0, 0)),
            pl.BlockSpec((d, h), lambda i: (0, 0)),
            pl.BlockSpec((1, h), lambda i: (0, 0)),
        ],
        out_specs=pl.BlockSpec((rows, h), lambda i: (i, 0)),
        out_shape=jax.ShapeDtypeStruct((n, h), jnp.float32),
    )(sums, cnts, nodes, w1, w2, b2)


def kernel(nodes, senders, receivers, W, b):
    n, d = nodes.shape
    send2d, recv2d = _pad_edges(
        senders.astype(jnp.int32), receivers.astype(jnp.int32), n)
    sums, cnts = _sc_segment_sum_count(nodes, send2d, recv2d)
    return _tc_finish(sums, cnts, nodes, W[:d], W[d:], b.reshape(1, -1))


# speedup vs baseline: 1.1238x; 1.1238x over previous
import functools

import jax
import jax.numpy as jnp
from jax import lax
from jax.experimental import pallas as pl
from jax.experimental.pallas import tpu as pltpu
from jax.experimental.pallas import tpu_sc as plsc

_CW = 128
_DUMP = 16
_HB = 128


def _sc_info():
    info = plsc.get_sparse_core_info()
    return info.num_cores, info.num_subcores


def _pad_edges(senders, receivers, n):
    e = senders.shape[0]
    nc, ns = _sc_info()
    nw = nc * ns
    quantum = nw * _CW * 4
    epad = ((e + quantum - 1) // quantum) * quantum
    pad = epad - e
    if pad:
        senders = jnp.concatenate(
            [senders, jnp.zeros((pad,), jnp.int32)])
        receivers = jnp.concatenate(
            [receivers, n + (jnp.arange(pad, dtype=jnp.int32) % _DUMP)])
    return senders.reshape(-1, _CW), receivers.reshape(-1, _CW)


def _sc_segment_sum_count(nodes, send2d, recv2d):
    n, d = nodes.shape
    nchunks = send2d.shape[0]
    nc, ns = _sc_info()
    nw = nc * ns
    cpw = nchunks // nw
    hcpw = cpw // 2
    hpairs = hcpw // 2
    assert hcpw % 2 == 0
    assert n + _DUMP <= _HB * _HB
    stripe = (n // (ns * 8)) * 8
    tail = n - ns * stripe
    nfull, rem = divmod(stripe, _CW)
    assert tail % 8 == 0 and tail + _DUMP <= _CW and rem % 8 == 0

    mesh = plsc.VectorSubcoreMesh(core_axis_name="c", subcore_axis_name="s")

    @functools.partial(
        pl.kernel,
        mesh=mesh,
        compiler_params=pltpu.CompilerParams(needs_layout_passes=False),
        out_type=[
            jax.ShapeDtypeStruct((nc, n, d), jnp.float32),
            jax.ShapeDtypeStruct((nc, _HB, _HB), jnp.float32),
        ],
        scratch_types=[
            pltpu.VMEM((hcpw, _CW), jnp.int32),
            pltpu.VMEM((hcpw, _CW), jnp.int32),
            pltpu.VMEM((_CW, d), jnp.float32),
            pltpu.VMEM((_CW, d), jnp.float32),
            pltpu.VMEM_SHARED((n + _DUMP, d), jnp.float32),
            pltpu.VMEM_SHARED((_HB, _HB), jnp.float32),
            pltpu.SemaphoreType.DMA,
            pltpu.SemaphoreType.DMA,
        ],
    )
    def sc_kernel(nodes_hbm, send_hbm, recv_hbm, sums_out, cnts_out,
                  idx_s, idx_r, rows0, rows1, acc, cnt_sh, sem0, sem1):
        c = lax.axis_index("c")
        s = lax.axis_index("s")
        wid = s * nc + c
        base_row = s * stripe

        def fill_zero(buf):
            def fill_row(i, _):
                for j in range(d // 16):
                    buf[i, pl.ds(j * 16, 16)] = jnp.zeros((16,), jnp.float32)
                return 0
            lax.fori_loop(0, _CW, fill_row, 0)

        def zero_acc():
            for r in range(nfull):
                pltpu.sync_copy(rows0,
                                acc.at[pl.ds(base_row + r * _CW, _CW)])
            if rem:
                pltpu.sync_copy(rows0.at[pl.ds(0, rem)],
                                acc.at[pl.ds(base_row + nfull * _CW, rem)])

            @pl.when(s == 0)
            def _zero_tail():
                pltpu.sync_copy(rows0.at[pl.ds(0, tail + _DUMP)],
                                acc.at[pl.ds(ns * stripe, tail + _DUMP)])

        fill_zero(rows0)
        zero_acc()
        plsc.subcore_barrier()

        for h in range(2):
            cbase = wid * cpw + h * hcpw
            pltpu.sync_copy(send_hbm.at[pl.ds(cbase, hcpw)], idx_s)
            pltpu.sync_copy(recv_hbm.at[pl.ds(cbase, hcpw)], idx_r)
            pltpu.async_copy(nodes_hbm.at[idx_s.at[0]], rows0, sem0)

            def pair_body(j, _):
                pltpu.make_async_copy(nodes_hbm.at[idx_s.at[0]], rows0,
                                      sem0).wait()
                pltpu.async_copy(nodes_hbm.at[idx_s.at[2 * j + 1]], rows1,
                                 sem1)
                pltpu.sync_copy(rows0, acc.at[idx_r.at[2 * j]], add=True)
                pltpu.make_async_copy(nodes_hbm.at[idx_s.at[0]], rows1,
                                      sem1).wait()

                @pl.when(j < hpairs - 1)
                def _prefetch():
                    pltpu.async_copy(nodes_hbm.at[idx_s.at[2 * j + 2]],
                                     rows0, sem0)

                pltpu.sync_copy(rows1, acc.at[idx_r.at[2 * j + 1]], add=True)
                return 0

            lax.fori_loop(0, hpairs, pair_body, 0)

        plsc.subcore_barrier()
        pltpu.sync_copy(acc.at[pl.ds(base_row, stripe)],
                        sums_out.at[c, pl.ds(base_row, stripe)])

        @pl.when(s == 0)
        def _copy_tail():
            pltpu.sync_copy(acc.at[pl.ds(ns * stripe, tail)],
                            sums_out.at[c, pl.ds(ns * stripe, tail)])

        plsc.subcore_barrier()

        fill_zero(rows0)

        @pl.when(s == 0)
        def _zero_cnt():
            pltpu.sync_copy(rows0, cnt_sh)

        plsc.subcore_barrier()

        ones = jnp.ones((16,), jnp.float32)
        for h in range(2):
            cbase = wid * cpw + h * hcpw
            pltpu.sync_copy(recv_hbm.at[pl.ds(cbase, hcpw)], idx_r)

            def hist_row(r, _):
                for j in range(_CW // 16):
                    v = idx_r[r, pl.ds(j * 16, 16)]
                    hi = lax.shift_right_logical(v, 7)
                    lo = lax.bitwise_and(v, 127)
                    plsc.addupdate_scatter(rows0, [hi, lo], ones)
                return 0

            lax.fori_loop(0, hcpw, hist_row, 0)

        base16 = lax.iota(jnp.int32, 16)
        for j in range(_CW // 16):
            idx_r[0, pl.ds(j * 16, 16)] = base16 + j * 16
        pltpu.sync_copy(rows0, cnt_sh.at[idx_r.at[0]], add=True)
        plsc.subcore_barrier()

        @pl.when(s == 0)
        def _copy_cnt():
            pltpu.sync_copy(cnt_sh, cnts_out.at[c])

    return sc_kernel(nodes, send2d, recv2d)


def _tc_finish(sums, cnts_col, nodes, w1, w2, b2):
    n, d = nodes.shape
    h = w1.shape[1]
    nc = sums.shape[0]
    rows = 2000
    grid = (n // rows,)

    def tc_kernel(sums_ref, cnts_ref, nodes_ref, w1_ref, w2_ref, b_ref,
                  out_ref):
        ssum = sums_ref[0]
        csum = cnts_ref[0]
        for i in range(1, nc):
            ssum = ssum + sums_ref[i]
            csum = csum + cnts_ref[i]
        he = ssum / jnp.maximum(csum, 1.0)
        acc = jnp.dot(he, w1_ref[...], preferred_element_type=jnp.float32)
        acc = acc + jnp.dot(nodes_ref[...], w2_ref[...],
                            preferred_element_type=jnp.float32)
        out_ref[...] = jnp.maximum(acc + b_ref[...], 0.0)

    return pl.pallas_call(
        tc_kernel,
        grid=grid,
        in_specs=[
            pl.BlockSpec((nc, rows, d), lambda i: (0, i, 0)),
            pl.BlockSpec((nc, rows, 1), lambda i: (0, i, 0)),
            pl.BlockSpec((rows, d), lambda i: (i, 0)),
            pl.BlockSpec((d, h), lambda i: (0, 0)),
            pl.BlockSpec((d, h), lambda i: (0, 0)),
            pl.BlockSpec((1, h), lambda i: (0, 0)),
        ],
        out_specs=pl.BlockSpec((rows, h), lambda i: (i, 0)),
        out_shape=jax.ShapeDtypeStruct((n, h), jnp.float32),
    )(sums, cnts_col, nodes, w1, w2, b2)


def kernel(nodes, senders, receivers, W, b):
    n, d = nodes.shape
    send2d, recv2d = _pad_edges(
        senders.astype(jnp.int32), receivers.astype(jnp.int32), n)
    sums, cnts = _sc_segment_sum_count(nodes, send2d, recv2d)
    nc = cnts.shape[0]
    cnts_col = cnts.reshape(nc, _HB * _HB)[:, :n, None]
    return _tc_finish(sums, cnts_col, nodes, W[:d], W[d:], b.reshape(1, -1))


# speedup vs baseline: 1.1797x; 1.0497x over previous
import functools

import jax
import jax.numpy as jnp
from jax import lax
from jax.experimental import pallas as pl
from jax.experimental.pallas import tpu as pltpu
from jax.experimental.pallas import tpu_sc as plsc

_CW = 128
_DUMP = 16
_HB = 128


def _sc_info():
    info = plsc.get_sparse_core_info()
    return info.num_cores, info.num_subcores


def _pad_edges(senders, receivers, n):
    e = senders.shape[0]
    nc, ns = _sc_info()
    nw = nc * ns
    quantum = nw * _CW * 4
    epad = ((e + quantum - 1) // quantum) * quantum
    pad = epad - e
    if pad:
        senders = jnp.concatenate(
            [senders, jnp.zeros((pad,), jnp.int32)])
        receivers = jnp.concatenate(
            [receivers, n + (jnp.arange(pad, dtype=jnp.int32) % _DUMP)])
    return senders.reshape(-1, _CW), receivers.reshape(-1, _CW)


def _sc_segment_sum_count(nodes, send2d, recv2d):
    n, d = nodes.shape
    nchunks = send2d.shape[0]
    nc, ns = _sc_info()
    nw = nc * ns
    cpw = nchunks // nw
    hcpw = cpw // 2
    hpairs = hcpw // 2
    assert hcpw % 2 == 0
    assert n + _DUMP <= _HB * _HB
    stripe = (n // (ns * 8)) * 8
    tail = n - ns * stripe
    nfull, rem = divmod(stripe, _CW)
    assert tail % 8 == 0 and tail + _DUMP <= _CW and rem % 8 == 0

    mesh = plsc.VectorSubcoreMesh(core_axis_name="c", subcore_axis_name="s")

    @functools.partial(
        pl.kernel,
        mesh=mesh,
        compiler_params=pltpu.CompilerParams(needs_layout_passes=False),
        out_type=[
            jax.ShapeDtypeStruct((nc, n, d), jnp.float32),
            jax.ShapeDtypeStruct((nc, _HB, _HB), jnp.float32),
        ],
        scratch_types=[
            pltpu.VMEM((hcpw, _CW), jnp.int32),
            pltpu.VMEM((hcpw, _CW), jnp.int32),
            pltpu.VMEM((2 * _CW, d), jnp.float32),
            pltpu.VMEM_SHARED((n + _DUMP, d), jnp.float32),
            pltpu.VMEM_SHARED((_HB, _HB), jnp.float32),
            pltpu.SemaphoreType.DMA,
            pltpu.SemaphoreType.DMA,
            pltpu.SemaphoreType.DMA,
            pltpu.SemaphoreType.DMA,
        ],
    )
    def sc_kernel(nodes_hbm, send_hbm, recv_hbm, sums_out, cnts_out,
                  idx_s, idx_r, rows, acc, cnt_sh, semA, semB, semC, semD):
        c = lax.axis_index("c")
        s = lax.axis_index("s")
        wid = s * nc + c
        base_row = s * stripe

        def fill_zero():
            def fill_row(i, _):
                for j in range(d // 16):
                    rows[i, pl.ds(j * 16, 16)] = jnp.zeros((16,), jnp.float32)
                return 0
            lax.fori_loop(0, _CW, fill_row, 0)

        zrow = rows.at[pl.ds(0, _CW)]

        def zero_acc():
            for r in range(nfull):
                pltpu.sync_copy(zrow,
                                acc.at[pl.ds(base_row + r * _CW, _CW)])
            if rem:
                pltpu.sync_copy(rows.at[pl.ds(0, rem)],
                                acc.at[pl.ds(base_row + nfull * _CW, rem)])

            @pl.when(s == 0)
            def _zero_tail():
                pltpu.sync_copy(rows.at[pl.ds(0, tail + _DUMP)],
                                acc.at[pl.ds(ns * stripe, tail + _DUMP)])

        fill_zero()
        zero_acc()
        plsc.subcore_barrier()

        for h in range(2):
            cbase = wid * cpw + h * hcpw
            pltpu.sync_copy(send_hbm.at[pl.ds(cbase, hcpw)], idx_s)
            pltpu.sync_copy(recv_hbm.at[pl.ds(cbase, hcpw)], idx_r)

            def fire_pair(chunk, q0, sa, sb):
                pltpu.async_copy(nodes_hbm.at[idx_s.at[chunk, pl.ds(0, 64)]],
                                 rows.at[pl.ds(q0 * 64, 64)], sa)
                pltpu.async_copy(nodes_hbm.at[idx_s.at[chunk, pl.ds(64, 64)]],
                                 rows.at[pl.ds(q0 * 64 + 64, 64)], sb)

            def drain(q, sem):
                pltpu.make_async_copy(
                    nodes_hbm.at[idx_s.at[0, pl.ds(0, 64)]],
                    rows.at[pl.ds(q * 64, 64)], sem).wait()

            fire_pair(0, 0, semA, semB)
            fire_pair(1, 2, semC, semD)

            def pair_body(j, _):
                drain(0, semA)
                drain(1, semB)
                pltpu.sync_copy(rows.at[pl.ds(0, _CW)],
                                acc.at[idx_r.at[2 * j]], add=True)

                @pl.when(j < hpairs - 1)
                def _f0():
                    fire_pair(2 * j + 2, 0, semA, semB)

                drain(2, semC)
                drain(3, semD)
                pltpu.sync_copy(rows.at[pl.ds(_CW, _CW)],
                                acc.at[idx_r.at[2 * j + 1]], add=True)

                @pl.when(j < hpairs - 1)
                def _f1():
                    fire_pair(2 * j + 3, 2, semC, semD)

                return 0

            lax.fori_loop(0, hpairs, pair_body, 0)

        plsc.subcore_barrier()
        pltpu.sync_copy(acc.at[pl.ds(base_row, stripe)],
                        sums_out.at[c, pl.ds(base_row, stripe)])

        @pl.when(s == 0)
        def _copy_tail():
            pltpu.sync_copy(acc.at[pl.ds(ns * stripe, tail)],
                            sums_out.at[c, pl.ds(ns * stripe, tail)])

        plsc.subcore_barrier()

        fill_zero()

        @pl.when(s == 0)
        def _zero_cnt():
            pltpu.sync_copy(zrow, cnt_sh)

        plsc.subcore_barrier()

        ones = jnp.ones((16,), jnp.float32)
        for h in range(2):
            cbase = wid * cpw + h * hcpw
            pltpu.sync_copy(recv_hbm.at[pl.ds(cbase, hcpw)], idx_r)

            def hist_row(r, _):
                for j in range(_CW // 16):
                    v = idx_r[r, pl.ds(j * 16, 16)]
                    hi = lax.shift_right_logical(v, 7)
                    lo = lax.bitwise_and(v, 127)
                    plsc.addupdate_scatter(rows, [hi, lo], ones)
                return 0

            lax.fori_loop(0, hcpw, hist_row, 0)

        base16 = lax.iota(jnp.int32, 16)
        for j in range(_CW // 16):
            idx_r[0, pl.ds(j * 16, 16)] = base16 + j * 16
        pltpu.sync_copy(zrow, cnt_sh.at[idx_r.at[0]], add=True)
        plsc.subcore_barrier()

        @pl.when(s == 0)
        def _copy_cnt():
            pltpu.sync_copy(cnt_sh, cnts_out.at[c])

    return sc_kernel(nodes, send2d, recv2d)


def _tc_finish(sums, cnts_col, nodes, w1, w2, b2):
    n, d = nodes.shape
    h = w1.shape[1]
    nc = sums.shape[0]
    rows = 2000
    grid = (n // rows,)

    def tc_kernel(sums_ref, cnts_ref, nodes_ref, w1_ref, w2_ref, b_ref,
                  out_ref):
        ssum = sums_ref[0]
        csum = cnts_ref[0]
        for i in range(1, nc):
            ssum = ssum + sums_ref[i]
            csum = csum + cnts_ref[i]
        he = ssum / jnp.maximum(csum, 1.0)
        acc = jnp.dot(he, w1_ref[...], preferred_element_type=jnp.float32)
        acc = acc + jnp.dot(nodes_ref[...], w2_ref[...],
                            preferred_element_type=jnp.float32)
        out_ref[...] = jnp.maximum(acc + b_ref[...], 0.0)

    return pl.pallas_call(
        tc_kernel,
        grid=grid,
        in_specs=[
            pl.BlockSpec((nc, rows, d), lambda i: (0, i, 0)),
            pl.BlockSpec((nc, rows, 1), lambda i: (0, i, 0)),
            pl.BlockSpec((rows, d), lambda i: (i, 0)),
            pl.BlockSpec((d, h), lambda i: (0, 0)),
            pl.BlockSpec((d, h), lambda i: (0, 0)),
            pl.BlockSpec((1, h), lambda i: (0, 0)),
        ],
        out_specs=pl.BlockSpec((rows, h), lambda i: (i, 0)),
        out_shape=jax.ShapeDtypeStruct((n, h), jnp.float32),
    )(sums, cnts_col, nodes, w1, w2, b2)


def kernel(nodes, senders, receivers, W, b):
    n, d = nodes.shape
    send2d, recv2d = _pad_edges(
        senders.astype(jnp.int32), receivers.astype(jnp.int32), n)
    sums, cnts = _sc_segment_sum_count(nodes, send2d, recv2d)
    nc = cnts.shape[0]
    cnts_col = cnts.reshape(nc, _HB * _HB)[:, :n, None]
    return _tc_finish(sums, cnts_col, nodes, W[:d], W[d:], b.reshape(1, -1))


# speedup vs baseline: 1.1802x; 1.0005x over previous
import functools

import jax
import jax.numpy as jnp
from jax import lax
from jax.experimental import pallas as pl
from jax.experimental.pallas import tpu as pltpu
from jax.experimental.pallas import tpu_sc as plsc

_CW = 128
_DUMP = 16
_HB = 128


def _sc_info():
    info = plsc.get_sparse_core_info()
    return info.num_cores, info.num_subcores


def _pad_edges(senders, receivers, n):
    e = senders.shape[0]
    nc, ns = _sc_info()
    nw = nc * ns
    quantum = nw * _CW * 4
    epad = ((e + quantum - 1) // quantum) * quantum
    pad = epad - e
    if pad:
        senders = jnp.concatenate(
            [senders, jnp.zeros((pad,), jnp.int32)])
        receivers = jnp.concatenate(
            [receivers, n + (jnp.arange(pad, dtype=jnp.int32) % _DUMP)])
    return senders.reshape(-1, _CW), receivers.reshape(-1, _CW)


def _sc_segment_sum_count(nodes, send2d, recv2d):
    n, d = nodes.shape
    nchunks = send2d.shape[0]
    nc, ns = _sc_info()
    nw = nc * ns
    cpw = nchunks // nw
    hcpw = cpw // 2
    hpairs = hcpw // 2
    assert hcpw % 2 == 0
    assert n + _DUMP <= _HB * _HB
    stripe = (n // (ns * 8)) * 8
    tail = n - ns * stripe
    nfull, rem = divmod(stripe, _CW)
    assert tail % 8 == 0 and tail + _DUMP <= _CW and rem % 8 == 0

    mesh = plsc.VectorSubcoreMesh(core_axis_name="c", subcore_axis_name="s")

    @functools.partial(
        pl.kernel,
        mesh=mesh,
        compiler_params=pltpu.CompilerParams(needs_layout_passes=False),
        out_type=[
            jax.ShapeDtypeStruct((nc, n, d), jnp.float32),
            jax.ShapeDtypeStruct((nc, _HB, _HB), jnp.float32),
        ],
        scratch_types=[
            pltpu.VMEM((hcpw, _CW), jnp.int32),
            pltpu.VMEM((hcpw, _CW), jnp.int32),
            pltpu.VMEM((2 * _CW, d), jnp.float32),
            pltpu.VMEM_SHARED((n + _DUMP, d), jnp.float32),
            pltpu.VMEM_SHARED((_HB, _HB), jnp.float32),
            [pltpu.SemaphoreType.DMA] * 8,
        ],
    )
    def sc_kernel(nodes_hbm, send_hbm, recv_hbm, sums_out, cnts_out,
                  idx_s, idx_r, rows, acc, cnt_sh, sems):
        c = lax.axis_index("c")
        s = lax.axis_index("s")
        wid = s * nc + c
        base_row = s * stripe

        def fill_zero():
            def fill_row(i, _):
                for j in range(d // 16):
                    rows[i, pl.ds(j * 16, 16)] = jnp.zeros((16,), jnp.float32)
                return 0
            lax.fori_loop(0, _CW, fill_row, 0)

        zrow = rows.at[pl.ds(0, _CW)]

        def zero_acc():
            for r in range(nfull):
                pltpu.sync_copy(zrow,
                                acc.at[pl.ds(base_row + r * _CW, _CW)])
            if rem:
                pltpu.sync_copy(rows.at[pl.ds(0, rem)],
                                acc.at[pl.ds(base_row + nfull * _CW, rem)])

            @pl.when(s == 0)
            def _zero_tail():
                pltpu.sync_copy(rows.at[pl.ds(0, tail + _DUMP)],
                                acc.at[pl.ds(ns * stripe, tail + _DUMP)])

        fill_zero()
        zero_acc()
        plsc.subcore_barrier()

        for h in range(2):
            cbase = wid * cpw + h * hcpw
            pltpu.sync_copy(send_hbm.at[pl.ds(cbase, hcpw)], idx_s)
            pltpu.sync_copy(recv_hbm.at[pl.ds(cbase, hcpw)], idx_r)

            def fire_quad(chunk, half):
                for p in range(4):
                    pltpu.async_copy(
                        nodes_hbm.at[idx_s.at[chunk, pl.ds(p * 32, 32)]],
                        rows.at[pl.ds(half * _CW + p * 32, 32)],
                        sems[half * 4 + p])

            def drain_quad(half):
                for p in range(4):
                    pltpu.make_async_copy(
                        nodes_hbm.at[idx_s.at[0, pl.ds(0, 32)]],
                        rows.at[pl.ds(half * _CW + p * 32, 32)],
                        sems[half * 4 + p]).wait()

            fire_quad(0, 0)
            fire_quad(1, 1)

            def pair_body(j, _):
                drain_quad(0)
                pltpu.sync_copy(rows.at[pl.ds(0, _CW)],
                                acc.at[idx_r.at[2 * j]], add=True)

                @pl.when(j < hpairs - 1)
                def _f0():
                    fire_quad(2 * j + 2, 0)

                drain_quad(1)
                pltpu.sync_copy(rows.at[pl.ds(_CW, _CW)],
                                acc.at[idx_r.at[2 * j + 1]], add=True)

                @pl.when(j < hpairs - 1)
                def _f1():
                    fire_quad(2 * j + 3, 1)

                return 0

            lax.fori_loop(0, hpairs, pair_body, 0)

        plsc.subcore_barrier()
        pltpu.sync_copy(acc.at[pl.ds(base_row, stripe)],
                        sums_out.at[c, pl.ds(base_row, stripe)])

        @pl.when(s == 0)
        def _copy_tail():
            pltpu.sync_copy(acc.at[pl.ds(ns * stripe, tail)],
                            sums_out.at[c, pl.ds(ns * stripe, tail)])

        plsc.subcore_barrier()

        fill_zero()

        @pl.when(s == 0)
        def _zero_cnt():
            pltpu.sync_copy(zrow, cnt_sh)

        plsc.subcore_barrier()

        ones = jnp.ones((16,), jnp.float32)
        for h in range(2):
            cbase = wid * cpw + h * hcpw
            pltpu.sync_copy(recv_hbm.at[pl.ds(cbase, hcpw)], idx_r)

            def hist_row(r, _):
                for j in range(_CW // 16):
                    v = idx_r[r, pl.ds(j * 16, 16)]
                    hi = lax.shift_right_logical(v, 7)
                    lo = lax.bitwise_and(v, 127)
                    plsc.addupdate_scatter(rows, [hi, lo], ones)
                return 0

            lax.fori_loop(0, hcpw, hist_row, 0)

        base16 = lax.iota(jnp.int32, 16)
        for j in range(_CW // 16):
            idx_r[0, pl.ds(j * 16, 16)] = base16 + j * 16
        pltpu.sync_copy(zrow, cnt_sh.at[idx_r.at[0]], add=True)
        plsc.subcore_barrier()

        @pl.when(s == 0)
        def _copy_cnt():
            pltpu.sync_copy(cnt_sh, cnts_out.at[c])

    return sc_kernel(nodes, send2d, recv2d)


def _tc_finish(sums, cnts_col, nodes, w1, w2, b2):
    n, d = nodes.shape
    h = w1.shape[1]
    nc = sums.shape[0]
    rows = 2000
    grid = (n // rows,)

    def tc_kernel(sums_ref, cnts_ref, nodes_ref, w1_ref, w2_ref, b_ref,
                  out_ref):
        ssum = sums_ref[0]
        csum = cnts_ref[0]
        for i in range(1, nc):
            ssum = ssum + sums_ref[i]
            csum = csum + cnts_ref[i]
        he = ssum / jnp.maximum(csum, 1.0)
        acc = jnp.dot(he, w1_ref[...], preferred_element_type=jnp.float32)
        acc = acc + jnp.dot(nodes_ref[...], w2_ref[...],
                            preferred_element_type=jnp.float32)
        out_ref[...] = jnp.maximum(acc + b_ref[...], 0.0)

    return pl.pallas_call(
        tc_kernel,
        grid=grid,
        in_specs=[
            pl.BlockSpec((nc, rows, d), lambda i: (0, i, 0)),
            pl.BlockSpec((nc, rows, 1), lambda i: (0, i, 0)),
            pl.BlockSpec((rows, d), lambda i: (i, 0)),
            pl.BlockSpec((d, h), lambda i: (0, 0)),
            pl.BlockSpec((d, h), lambda i: (0, 0)),
            pl.BlockSpec((1, h), lambda i: (0, 0)),
        ],
        out_specs=pl.BlockSpec((rows, h), lambda i: (i, 0)),
        out_shape=jax.ShapeDtypeStruct((n, h), jnp.float32),
    )(sums, cnts_col, nodes, w1, w2, b2)


def kernel(nodes, senders, receivers, W, b):
    n, d = nodes.shape
    send2d, recv2d = _pad_edges(
        senders.astype(jnp.int32), receivers.astype(jnp.int32), n)
    sums, cnts = _sc_segment_sum_count(nodes, send2d, recv2d)
    nc = cnts.shape[0]
    cnts_col = cnts.reshape(nc, _HB * _HB)[:, :n, None]
    return _tc_finish(sums, cnts_col, nodes, W[:d], W[d:], b.reshape(1, -1))
